# Initial kernel scaffold; baseline (speedup 1.0000x reference)
import jax
import jax.numpy as jnp
from jax.experimental import pallas as pl


def kernel(voxel_features, voxel_coords, batch_size, params):
    raise NotImplementedError("write your pallas kernel here")


# speedup vs baseline: 1.1162x; 1.1162x over previous
import math

import numpy as np

import jax
import jax.numpy as jnp
from jax.experimental import pallas as pl

_BNS = 1.0 / math.sqrt(1.0 + 1e-3)
_SPATIAL = (25, 64, 64)
_B = 2
_YB = 8


def _pcall(body, out_shape, interpret=False):
    return pl.pallas_call(body, out_shape=out_shape, interpret=interpret)


def _fold_w(w, gi, go, sy, py):
    tz, ty, tx, ci, co = w.shape
    P = np.zeros((3, gi, go, ty), np.float32)
    for si, s in enumerate((-1, 0, 1)):
        for g_in in range(gi):
            for g_out in range(go):
                dy = g_in + s * gi - sy * g_out + py
                if 0 <= dy < ty:
                    P[si, g_in, g_out, dy] = 1.0
    wf = jnp.einsum('sghy,zyxio->szxgiho', P, w)
    wf = wf.reshape(3, tz, tx, gi * ci, go * co)
    s_active = [si for si in range(3) if P[si].any()]
    return wf, s_active


def _tap(xp_ref, si, dz, dx, sz, sx, zo, xo, k):
    xs = xp_ref[pl.ds(dz, sz * zo), pl.ds(si, _YB), pl.ds(dx, sx * xo), :]
    if sz > 1 or sx > 1:
        xs = xs.reshape(zo, sz, _YB, xo, sx, k)[:, 0, :, :, 0, :]
    return xs


def _fconv(xp, wf, s_active, sz, sx, zo, xo, maskx=None, identity=None,
           dilate=False):
    _, tz, tx, k, n = wf.shape
    m = zo * _YB * xo

    def body(*refs):
        xp_ref, wf_ref = refs[0], refs[1]
        i = 2
        maskx_ref = None
        id_ref = None
        if maskx is not None:
            maskx_ref = refs[i]
            i += 1
        if identity is not None:
            id_ref = refs[i]
            i += 1
        out_ref = refs[-1]

        n_taps = len(s_active) * tz * tx
        s0 = s_active[0]

        def tap_body(t, acc):
            j = t // (tz * tx)
            dz = (t // tx) % tz
            dx = t % tx
            si = j + s0
            xs = _tap(xp_ref, si, dz, dx, sz, sx, zo, xo, k)
            return acc + jnp.dot(xs.reshape(m, k), wf_ref[si, dz, dx],
                                 preferred_element_type=jnp.float32)

        acc = jax.lax.fori_loop(0, n_taps, tap_body,
                                jnp.zeros((m, n), jnp.float32))
        if dilate:
            out_ref[...] = (acc > 0.0).astype(jnp.float32).reshape(zo, _YB, xo, n)
            return
        if maskx_ref is not None:
            acc = acc * maskx_ref[...].reshape(m, n)
        acc = acc * _BNS
        if id_ref is not None:
            acc = acc + id_ref[...].reshape(m, n)
        out_ref[...] = jnp.maximum(acc, 0.0).reshape(zo, _YB, xo, n)

    args = [xp, wf]
    if maskx is not None:
        args.append(maskx)
    if identity is not None:
        args.append(identity)
    return _pcall(body, jax.ShapeDtypeStruct((zo, _YB, xo, n), jnp.float32))(*args)


def _padzx(x, pz, px):
    return jnp.pad(x, (pz, (1, 1), px, (0, 0)))


def _subm_args(w, g, c):
    wf, s_act = _fold_w(w.reshape(3, 3, 3, c, c), g, g, 1, 1)
    return wf, s_act


def _block(h, w1, w2, maskx, g, c, zo, xo):
    wf1, sa1 = _subm_args(w1, g, c)
    wf2, sa2 = _subm_args(w2, g, c)
    o = _fconv(_padzx(h, (1, 1), (1, 1)), wf1, sa1, 1, 1, zo, xo, maskx=maskx)
    return _fconv(_padzx(o, (1, 1), (1, 1)), wf2, sa2, 1, 1, zo, xo,
                  maskx=maskx, identity=h)


def kernel(voxel_features, voxel_coords, batch_size, params):
    Z, Y, X = _SPATIAL
    N = voxel_features.shape[0]
    p = params

    b = voxel_coords[:, 0] % batch_size
    z = voxel_coords[:, 1] % Z
    y = voxel_coords[:, 2] % Y
    x = voxel_coords[:, 3] % X

    feats8 = jnp.concatenate(
        [voxel_features,
         jnp.ones((N, 1), jnp.float32),
         jnp.zeros((N, 2), jnp.float32)], axis=1)
    dense8 = jnp.zeros((_B, Z, Y, X, 8), jnp.float32).at[b, z, y, x].set(feats8)
    dense8f = dense8.reshape(_B, Z, _YB, 8, X, 8).transpose(0, 1, 2, 4, 3, 5)
    dense8f = dense8f.reshape(_B, Z, _YB, X, 64)
    mask1f = dense8f[..., 5::8]
    mask1x = jnp.repeat(mask1f, 16, axis=-1)

    w_in8 = jnp.pad(p['w_in'], ((0, 0), (0, 0), (0, 0), (0, 3), (0, 0)))
    w_in_f, sa_in = _fold_w(w_in8, 8, 8, 1, 1)

    wd2, sa_d2 = _fold_w(p['w_d2'], 8, 4, 2, 1)
    wd3, sa_d3 = _fold_w(p['w_d3'], 4, 2, 2, 1)
    wd4, sa_d4 = _fold_w(p['w_d4'], 2, 1, 2, 1)
    wout, sa_out = _fold_w(p['w_out'], 1, 1, 1, 0)
    ones_w = np.ones((3, 3, 3, 1, 1), np.float32)
    dil2_w, sa_dil2 = _fold_w(jnp.asarray(ones_w), 8, 4, 2, 1)
    dil3_w, sa_dil3 = _fold_w(jnp.asarray(ones_w), 4, 2, 2, 1)
    dil4_w, sa_dil4 = _fold_w(jnp.asarray(ones_w), 2, 1, 2, 1)

    outs = []
    for bi in range(_B):
        h = _fconv(_padzx(dense8f[bi], (1, 1), (1, 1)), w_in_f, sa_in,
                   1, 1, 25, 64, maskx=mask1x[bi])
        h = _block(h, p['r1a1'], p['r1a2'], mask1x[bi], 8, 16, 25, 64)
        h = _block(h, p['r1b1'], p['r1b2'], mask1x[bi], 8, 16, 25, 64)

        h = _fconv(_padzx(h, (1, 2), (1, 1)), wd2, sa_d2, 2, 2, 13, 32)
        m2 = _fconv(_padzx(mask1f[bi], (1, 2), (1, 1)), dil2_w, sa_dil2,
                    2, 2, 13, 32, dilate=True)
        m2x = jnp.repeat(m2, 32, axis=-1)
        h = _block(h, p['r2a1'], p['r2a2'], m2x, 4, 32, 13, 32)
        h = _block(h, p['r2b1'], p['r2b2'], m2x, 4, 32, 13, 32)

        h = _fconv(_padzx(h, (1, 2), (1, 1)), wd3, sa_d3, 2, 2, 7, 16)
        m3 = _fconv(_padzx(m2, (1, 2), (1, 1)), dil3_w, sa_dil3,
                    2, 2, 7, 16, dilate=True)
        m3x = jnp.repeat(m3, 64, axis=-1)
        h = _block(h, p['r3a1'], p['r3a2'], m3x, 2, 64, 7, 16)
        h = _block(h, p['r3b1'], p['r3b2'], m3x, 2, 64, 7, 16)

        h = _fconv(_padzx(h, (0, 2), (1, 1)), wd4, sa_d4, 2, 2, 3, 8)
        m4 = _fconv(_padzx(m3, (0, 2), (1, 1)), dil4_w, sa_dil4,
                    2, 2, 3, 8, dilate=True)
        m4x = jnp.repeat(m4, 128, axis=-1)
        h = _block(h, p['r4a1'], p['r4a2'], m4x, 1, 128, 3, 8)
        h = _block(h, p['r4b1'], p['r4b2'], m4x, 1, 128, 3, 8)

        out = _fconv(jnp.pad(h, ((0, 1), (1, 1), (0, 0), (0, 0))), wout,
                     sa_out, 2, 1, 1, 8)
        outs.append(out)

    return jnp.stack(outs)


# speedup vs baseline: 1.2393x; 1.1103x over previous
import functools
import math

import numpy as np

import jax
import jax.numpy as jnp
from jax import lax
from jax.experimental import pallas as pl
from jax.experimental.pallas import tpu as pltpu
from jax.experimental.pallas import tpu_sc as plsc

_BNS = 1.0 / math.sqrt(1.0 + 1e-3)
_SPATIAL = (25, 64, 64)
_B = 2
_YB = 8


def _pcall(body, out_shape, interpret=False):
    return pl.pallas_call(body, out_shape=out_shape, interpret=interpret)


_NSITE = _B * 25 * 64 * 64
_TRASH = 128
_DR = 16


_NSH = _NSITE // 2
_HALF = _NSH + _TRASH


def _sc_scatter(idx0, idx1, feats16):
    NP = idx0.shape[0]
    ZB = 128
    CH = 512
    GR = 128
    n_chunks = NP // CH
    n_groups = CH // GR
    rows_t = _NSH // 16
    nz_full, nz_rem = divmod(rows_t, ZB)

    mesh = plsc.VectorSubcoreMesh(core_axis_name="c", subcore_axis_name="s")
    scratch = ([pltpu.VMEM_SHARED((_HALF, _DR), jnp.float32),
                pltpu.VMEM((ZB, _DR), jnp.float32)]
               + [pltpu.VMEM((GR,), jnp.int32) for _ in range(n_groups)]
               + [pltpu.VMEM((CH, _DR), jnp.float32),
                  pltpu.SemaphoreType.DMA])

    @functools.partial(pl.kernel, mesh=mesh,
                       out_type=jax.ShapeDtypeStruct((2 * _NSH, _DR),
                                                     jnp.float32),
                       scratch_types=scratch,
                       compiler_params=pltpu.CompilerParams(
                           use_tc_tiling_on_sc=False))
    def run(idx0_hbm, idx1_hbm, feats_hbm, out_hbm, shared, zbuf, *rest):
        idx_bufs = rest[:n_groups]
        rows_v = rest[n_groups]
        sem = rest[n_groups + 1]
        cid = lax.axis_index("c")
        sid = lax.axis_index("s")

        def zrow(r, carry):
            zbuf[r] = jnp.zeros((_DR,), jnp.float32)
            return carry
        lax.fori_loop(0, ZB, zrow, 0)
        base = sid * rows_t
        for k in range(nz_full):
            pltpu.sync_copy(zbuf, shared.at[pl.ds(base + k * ZB, ZB)])
        if nz_rem:
            pltpu.sync_copy(zbuf.at[pl.ds(0, nz_rem)],
                            shared.at[pl.ds(base + nz_full * ZB, nz_rem)])
        plsc.subcore_barrier()

        for c in range(2):
            @pl.when(jnp.logical_and(cid == c, sid == 0))
            def _scatter_phase(c=c):
                ih = idx0_hbm if c == 0 else idx1_hbm
                for ch in range(n_chunks):
                    pltpu.sync_copy(feats_hbm.at[pl.ds(ch * CH, CH)], rows_v)
                    for g in range(n_groups):
                        pltpu.sync_copy(ih.at[pl.ds(ch * CH + g * GR, GR)],
                                        idx_bufs[g])
                    for g in range(n_groups):
                        pltpu.async_copy(rows_v.at[pl.ds(g * GR, GR)],
                                         shared.at[idx_bufs[g]], sem).wait()
        plsc.subcore_barrier()
        pltpu.sync_copy(shared.at[pl.ds(base, rows_t)],
                        out_hbm.at[pl.ds(cid * _NSH + base, rows_t)])

    return run(idx0, idx1, feats16)


def _fold_w(w, gi, go, sy, py):
    tz, ty, tx, ci, co = w.shape
    P = np.zeros((3, gi, go, ty), np.float32)
    for si, s in enumerate((-1, 0, 1)):
        for g_in in range(gi):
            for g_out in range(go):
                dy = g_in + s * gi - sy * g_out + py
                if 0 <= dy < ty:
                    P[si, g_in, g_out, dy] = 1.0
    wf = jnp.einsum('sghy,zyxio->szxgiho', P, w)
    wf = wf.reshape(3, tz, tx, gi * ci, go * co)
    s_active = [si for si in range(3) if P[si].any()]
    return wf, s_active


def _tap(xp_ref, si, dz, dx, sz, sx, zo, xo, k):
    xs = xp_ref[pl.ds(dz, sz * zo), pl.ds(si, _YB), pl.ds(dx, sx * xo), :]
    if sz > 1 or sx > 1:
        xs = xs.reshape(zo, sz, _YB, xo, sx, k)[:, 0, :, :, 0, :]
    return xs


def _fconv(xp, wf, s_active, sz, sx, zo, xo, maskx=None, identity=None,
           dilate=False):
    _, tz, tx, k, n = wf.shape
    m = zo * _YB * xo

    def body(*refs):
        xp_ref, wf_ref = refs[0], refs[1]
        i = 2
        maskx_ref = None
        id_ref = None
        if maskx is not None:
            maskx_ref = refs[i]
            i += 1
        if identity is not None:
            id_ref = refs[i]
            i += 1
        out_ref = refs[-1]

        n_taps = len(s_active) * tz * tx
        s0 = s_active[0]

        def tap_body(t, acc):
            j = t // (tz * tx)
            dz = (t // tx) % tz
            dx = t % tx
            si = j + s0
            xs = _tap(xp_ref, si, dz, dx, sz, sx, zo, xo, k)
            return acc + jnp.dot(xs.reshape(m, k), wf_ref[si, dz, dx],
                                 preferred_element_type=jnp.float32)

        acc = jax.lax.fori_loop(0, n_taps, tap_body,
                                jnp.zeros((m, n), jnp.float32))
        if dilate:
            out_ref[...] = (acc > 0.0).astype(jnp.float32).reshape(zo, _YB, xo, n)
            return
        if maskx_ref is not None:
            acc = acc * maskx_ref[...].reshape(m, n)
        acc = acc * _BNS
        if id_ref is not None:
            acc = acc + id_ref[...].reshape(m, n)
        out_ref[...] = jnp.maximum(acc, 0.0).reshape(zo, _YB, xo, n)

    args = [xp, wf]
    if maskx is not None:
        args.append(maskx)
    if identity is not None:
        args.append(identity)
    return _pcall(body, jax.ShapeDtypeStruct((zo, _YB, xo, n), jnp.float32))(*args)


def _padzx(x, pz, px):
    return jnp.pad(x, (pz, (1, 1), px, (0, 0)))


def _subm_args(w, g, c):
    wf, s_act = _fold_w(w.reshape(3, 3, 3, c, c), g, g, 1, 1)
    return wf, s_act


def _block(h, w1, w2, maskx, g, c, zo, xo):
    wf1, sa1 = _subm_args(w1, g, c)
    wf2, sa2 = _subm_args(w2, g, c)
    o = _fconv(_padzx(h, (1, 1), (1, 1)), wf1, sa1, 1, 1, zo, xo, maskx=maskx)
    return _fconv(_padzx(o, (1, 1), (1, 1)), wf2, sa2, 1, 1, zo, xo,
                  maskx=maskx, identity=h)


def kernel(voxel_features, voxel_coords, batch_size, params):
    Z, Y, X = _SPATIAL
    N = voxel_features.shape[0]
    p = params

    b = voxel_coords[:, 0] % batch_size
    z = voxel_coords[:, 1] % Z
    y = voxel_coords[:, 2] % Y
    x = voxel_coords[:, 3] % X

    feats16 = jnp.concatenate(
        [voxel_features,
         jnp.ones((N, 1), jnp.float32),
         jnp.zeros((N, _DR - 6), jnp.float32)], axis=1)
    idx = ((((b * Z + z) * _YB + y // 8) * X + x) * 8 + y % 8).astype(jnp.int32)
    npad = (-N) % 512
    pad_i = jnp.full((npad,), _NSH, jnp.int32)
    idx0 = jnp.concatenate(
        [jnp.where(idx < _NSH, idx, _NSH).astype(jnp.int32), pad_i])
    idx1 = jnp.concatenate(
        [jnp.where(idx >= _NSH, idx - _NSH, _NSH).astype(jnp.int32), pad_i])
    feats_p = jnp.concatenate([feats16, jnp.zeros((npad, _DR), jnp.float32)])
    table = _sc_scatter(idx0, idx1, feats_p)
    dense8f = table.reshape(_B, Z, _YB, X, 128)
    mask1f = dense8f[..., 5::16]
    mask1x = jnp.repeat(mask1f, 16, axis=-1)

    w_in16 = jnp.pad(p['w_in'], ((0, 0), (0, 0), (0, 0), (0, 11), (0, 0)))
    w_in_f, sa_in = _fold_w(w_in16, 8, 8, 1, 1)

    wd2, sa_d2 = _fold_w(p['w_d2'], 8, 4, 2, 1)
    wd3, sa_d3 = _fold_w(p['w_d3'], 4, 2, 2, 1)
    wd4, sa_d4 = _fold_w(p['w_d4'], 2, 1, 2, 1)
    wout, sa_out = _fold_w(p['w_out'], 1, 1, 1, 0)
    ones_w = np.ones((3, 3, 3, 1, 1), np.float32)
    dil2_w, sa_dil2 = _fold_w(jnp.asarray(ones_w), 8, 4, 2, 1)
    dil3_w, sa_dil3 = _fold_w(jnp.asarray(ones_w), 4, 2, 2, 1)
    dil4_w, sa_dil4 = _fold_w(jnp.asarray(ones_w), 2, 1, 2, 1)

    outs = []
    for bi in range(_B):
        h = _fconv(_padzx(dense8f[bi], (1, 1), (1, 1)), w_in_f, sa_in,
                   1, 1, 25, 64, maskx=mask1x[bi])
        h = _block(h, p['r1a1'], p['r1a2'], mask1x[bi], 8, 16, 25, 64)
        h = _block(h, p['r1b1'], p['r1b2'], mask1x[bi], 8, 16, 25, 64)

        h = _fconv(_padzx(h, (1, 2), (1, 1)), wd2, sa_d2, 2, 2, 13, 32)
        m2 = _fconv(_padzx(mask1f[bi], (1, 2), (1, 1)), dil2_w, sa_dil2,
                    2, 2, 13, 32, dilate=True)
        m2x = jnp.repeat(m2, 32, axis=-1)
        h = _block(h, p['r2a1'], p['r2a2'], m2x, 4, 32, 13, 32)
        h = _block(h, p['r2b1'], p['r2b2'], m2x, 4, 32, 13, 32)

        h = _fconv(_padzx(h, (1, 2), (1, 1)), wd3, sa_d3, 2, 2, 7, 16)
        m3 = _fconv(_padzx(m2, (1, 2), (1, 1)), dil3_w, sa_dil3,
                    2, 2, 7, 16, dilate=True)
        m3x = jnp.repeat(m3, 64, axis=-1)
        h = _block(h, p['r3a1'], p['r3a2'], m3x, 2, 64, 7, 16)
        h = _block(h, p['r3b1'], p['r3b2'], m3x, 2, 64, 7, 16)

        h = _fconv(_padzx(h, (0, 2), (1, 1)), wd4, sa_d4, 2, 2, 3, 8)
        m4 = _fconv(_padzx(m3, (0, 2), (1, 1)), dil4_w, sa_dil4,
                    2, 2, 3, 8, dilate=True)
        m4x = jnp.repeat(m4, 128, axis=-1)
        h = _block(h, p['r4a1'], p['r4a2'], m4x, 1, 128, 3, 8)
        h = _block(h, p['r4b1'], p['r4b2'], m4x, 1, 128, 3, 8)

        out = _fconv(jnp.pad(h, ((0, 1), (1, 1), (0, 0), (0, 0))), wout,
                     sa_out, 2, 1, 1, 8)
        outs.append(out)

    return jnp.stack(outs)


# speedup vs baseline: 1.3551x; 1.0935x over previous
import functools
import math

import numpy as np

import jax
import jax.numpy as jnp
from jax import lax
from jax.experimental import pallas as pl
from jax.experimental.pallas import tpu as pltpu
from jax.experimental.pallas import tpu_sc as plsc

_BNS = 1.0 / math.sqrt(1.0 + 1e-3)
_SPATIAL = (25, 64, 64)
_B = 2
_YB = 8


def _pcall(body, out_shape, scratch_shapes=(), interpret=False):
    return pl.pallas_call(body, out_shape=out_shape,
                          scratch_shapes=list(scratch_shapes),
                          interpret=interpret)


_NSITE = _B * 25 * 64 * 64
_TRASH = 128
_DR = 16
_NSH = _NSITE // 2
_HALF = _NSH + _TRASH


def _sc_scatter(idx0, idx1, feats16):
    NP = idx0.shape[0]
    ZB = 128
    CH = 512
    GR = 128
    n_chunks = NP // CH
    n_groups = CH // GR
    rows_t = _NSH // 16
    nz_full, nz_rem = divmod(rows_t, ZB)

    mesh = plsc.VectorSubcoreMesh(core_axis_name="c", subcore_axis_name="s")
    scratch = ([pltpu.VMEM_SHARED((_HALF, _DR), jnp.float32),
                pltpu.VMEM((ZB, _DR), jnp.float32)]
               + [pltpu.VMEM((GR,), jnp.int32) for _ in range(n_groups)]
               + [pltpu.VMEM((CH, _DR), jnp.float32),
                  pltpu.SemaphoreType.DMA])

    @functools.partial(pl.kernel, mesh=mesh,
                       out_type=jax.ShapeDtypeStruct((2 * _NSH, _DR),
                                                     jnp.float32),
                       scratch_types=scratch,
                       compiler_params=pltpu.CompilerParams(
                           use_tc_tiling_on_sc=False))
    def run(idx0_hbm, idx1_hbm, feats_hbm, out_hbm, shared, zbuf, *rest):
        idx_bufs = rest[:n_groups]
        rows_v = rest[n_groups]
        sem = rest[n_groups + 1]
        cid = lax.axis_index("c")
        sid = lax.axis_index("s")

        def zrow(r, carry):
            zbuf[r] = jnp.zeros((_DR,), jnp.float32)
            return carry
        lax.fori_loop(0, ZB, zrow, 0)
        base = sid * rows_t
        for k in range(nz_full):
            pltpu.sync_copy(zbuf, shared.at[pl.ds(base + k * ZB, ZB)])
        if nz_rem:
            pltpu.sync_copy(zbuf.at[pl.ds(0, nz_rem)],
                            shared.at[pl.ds(base + nz_full * ZB, nz_rem)])
        plsc.subcore_barrier()

        for c in range(2):
            @pl.when(jnp.logical_and(cid == c, sid == 0))
            def _scatter_phase(c=c):
                ih = idx0_hbm if c == 0 else idx1_hbm
                for ch in range(n_chunks):
                    pltpu.sync_copy(feats_hbm.at[pl.ds(ch * CH, CH)], rows_v)
                    for g in range(n_groups):
                        pltpu.sync_copy(ih.at[pl.ds(ch * CH + g * GR, GR)],
                                        idx_bufs[g])
                    for g in range(n_groups):
                        pltpu.async_copy(rows_v.at[pl.ds(g * GR, GR)],
                                         shared.at[idx_bufs[g]], sem).wait()
        plsc.subcore_barrier()
        pltpu.sync_copy(shared.at[pl.ds(base, rows_t)],
                        out_hbm.at[pl.ds(cid * _NSH + base, rows_t)])

    return run(idx0, idx1, feats16)


def _fold_w(w, gi, go, sy, py, scale=1.0):
    tz, ty, tx, ci, co = w.shape
    P = np.zeros((3, gi, go, ty), np.float32)
    for si, s in enumerate((-1, 0, 1)):
        for g_in in range(gi):
            for g_out in range(go):
                dy = g_in + s * gi - sy * g_out + py
                if 0 <= dy < ty:
                    P[si, g_in, g_out, dy] = scale
    wf = jnp.einsum('sghy,zyxio->szxgiho', P, w)
    wf = wf.reshape(3, tz, tx, gi * ci, go * co).astype(jnp.bfloat16)
    s_active = [si for si in range(3) if P[si].any()]
    return wf, s_active


def _taps(xp_ref, wf_ref, s_active, sz, sx, zo, xo, bz, bx, m, k, n, tz, tx):
    n_taps = len(s_active) * tz * tx
    s0 = s_active[0]

    def tap_body(t, acc):
        j = t // (tz * tx)
        dz = (t // tx) % tz
        dx = t % tx
        si = j + s0
        xs = xp_ref[pl.ds(dz + bz, sz * zo), pl.ds(si, _YB),
                    pl.ds(dx + bx, sx * xo), :]
        if sz > 1 or sx > 1:
            xs = xs.reshape(zo, sz, _YB, xo, sx, k)[:, 0, :, :, 0, :]
        return acc + jnp.dot(xs.reshape(m, k).astype(jnp.bfloat16),
                             wf_ref[si, dz, dx],
                             preferred_element_type=jnp.float32)

    return lax.fori_loop(0, n_taps, tap_body, jnp.zeros((m, n), jnp.float32))


def _store_padded(out_ref, val5, zo, xo):
    z = jnp.float32(0.0)
    out_ref[pl.ds(0, 1)] = jnp.broadcast_to(z, out_ref.shape)[0:1]
    out_ref[pl.ds(zo + 1, 2)] = jnp.broadcast_to(z, out_ref.shape)[:2]
    out_ref[:, pl.ds(0, 1)] = jnp.broadcast_to(z, out_ref.shape)[:, 0:1]
    out_ref[:, pl.ds(_YB + 1, 1)] = jnp.broadcast_to(z, out_ref.shape)[:, 0:1]
    out_ref[:, :, pl.ds(0, 1)] = jnp.broadcast_to(z, out_ref.shape)[:, :, 0:1]
    out_ref[:, :, pl.ds(xo + 1, 1)] = jnp.broadcast_to(
        z, out_ref.shape)[:, :, 0:1]
    out_ref[pl.ds(1, zo), pl.ds(1, _YB), pl.ds(1, xo), :] = val5


def _fconv(xp, wf, s_active, sz, sx, zo, xo, bz=0, bx=0, maskx=None,
           dilate=False, pad_out=True):
    _, tz, tx, k, n = wf.shape
    m = zo * _YB * xo

    def body(*refs):
        xp_ref, wf_ref = refs[0], refs[1]
        maskx_ref = refs[2] if maskx is not None else None
        out_ref = refs[-1]
        acc = _taps(xp_ref, wf_ref, s_active, sz, sx, zo, xo, bz, bx,
                    m, k, n, tz, tx)
        if dilate:
            val = (acc > 0.0).astype(jnp.float32)
        else:
            if maskx_ref is not None:
                acc = acc * maskx_ref[...].reshape(m, n)
            val = jnp.maximum(acc * _BNS, 0.0)
        val5 = val.reshape(zo, _YB, xo, n)
        if pad_out:
            _store_padded(out_ref, val5, zo, xo)
        else:
            out_ref[...] = val5

    out_shape = ((zo + 3, _YB + 2, xo + 2, n) if pad_out
                 else (zo, _YB, xo, n))
    args = [xp, wf]
    if maskx is not None:
        args.append(maskx)
    return _pcall(body, jax.ShapeDtypeStruct(out_shape, jnp.float32))(*args)


def _fblock(xp, wf1, wf2, sa, maskx, zo, xo):
    _, tz, tx, k, n = wf1.shape
    m = zo * _YB * xo

    def body(xp_ref, wf1_ref, wf2_ref, maskx_ref, out_ref):
        acc1 = _taps(xp_ref, wf1_ref, sa, 1, 1, zo, xo, 0, 0,
                     m, k, n, tz, tx)
        mask = maskx_ref[...].reshape(m, n)
        val1 = jnp.maximum(acc1 * mask * _BNS, 0.0).reshape(zo, _YB, xo, n)
        _store_padded(out_ref, val1, zo, xo)
        acc2 = _taps(out_ref, wf2_ref, sa, 1, 1, zo, xo, 0, 0,
                     m, k, n, tz, tx)
        ident = xp_ref[pl.ds(1, zo), pl.ds(1, _YB), pl.ds(1, xo), :]
        mask2 = maskx_ref[...].reshape(m, n)
        val2 = jnp.maximum(acc2 * mask2 * _BNS + ident.reshape(m, n), 0.0)
        _store_padded(out_ref, val2.reshape(zo, _YB, xo, n), zo, xo)

    shape = (zo + 3, _YB + 2, xo + 2, n)
    return _pcall(body, jax.ShapeDtypeStruct(shape, jnp.float32))(
        xp, wf1, wf2, maskx)


def _padzx(x):
    return jnp.pad(x, ((1, 2), (1, 1), (1, 1), (0, 0)))


def kernel(voxel_features, voxel_coords, batch_size, params):
    Z, Y, X = _SPATIAL
    N = voxel_features.shape[0]
    p = params

    b = voxel_coords[:, 0] % batch_size
    z = voxel_coords[:, 1] % Z
    y = voxel_coords[:, 2] % Y
    x = voxel_coords[:, 3] % X

    feats16 = jnp.concatenate(
        [voxel_features,
         jnp.ones((N, 1), jnp.float32),
         jnp.zeros((N, _DR - 6), jnp.float32)], axis=1)
    idx = ((((b * Z + z) * _YB + y // 8) * X + x) * 8 + y % 8).astype(jnp.int32)
    npad = (-N) % 512
    pad_i = jnp.full((npad,), _NSH, jnp.int32)
    idx0 = jnp.concatenate(
        [jnp.where(idx < _NSH, idx, _NSH).astype(jnp.int32), pad_i])
    idx1 = jnp.concatenate(
        [jnp.where(idx >= _NSH, idx - _NSH, _NSH).astype(jnp.int32), pad_i])
    feats_p = jnp.concatenate([feats16, jnp.zeros((npad, _DR), jnp.float32)])
    table = _sc_scatter(idx0, idx1, feats_p)
    dense8f = table.reshape(_B, Z, _YB, X, 128)
    mask1f = dense8f[..., 5::16]
    mask1x = jnp.repeat(mask1f, 16, axis=-1)

    w_in16 = jnp.pad(p['w_in'], ((0, 0), (0, 0), (0, 0), (0, 11), (0, 0)))
    w_in_f, sa_in = _fold_w(w_in16, 8, 8, 1, 1)

    fw = _fold_w
    wr = {k_: fw(p[k_].reshape(3, 3, 3, ci, ci), g, g, 1, 1)
          for k_, g, ci in [
              ('r1a1', 8, 16), ('r1a2', 8, 16), ('r1b1', 8, 16),
              ('r1b2', 8, 16),
              ('r2a1', 4, 32), ('r2a2', 4, 32), ('r2b1', 4, 32),
              ('r2b2', 4, 32),
              ('r3a1', 2, 64), ('r3a2', 2, 64), ('r3b1', 2, 64),
              ('r3b2', 2, 64),
              ('r4a1', 1, 128), ('r4a2', 1, 128), ('r4b1', 1, 128),
              ('r4b2', 1, 128)]}
    wd2, sa_d2 = fw(p['w_d2'], 8, 4, 2, 1)
    wd3, sa_d3 = fw(p['w_d3'], 4, 2, 2, 1)
    wd4, sa_d4 = fw(p['w_d4'], 2, 1, 2, 1)
    wout, sa_out = fw(p['w_out'], 1, 1, 1, 0)
    ones_w = jnp.asarray(np.ones((3, 3, 3, 1, 1), np.float32))
    dil2_w, sa_dil2 = _fold_w(ones_w, 8, 4, 2, 1)
    dil3_w, sa_dil3 = _fold_w(ones_w, 4, 2, 2, 1)
    dil4_w, sa_dil4 = _fold_w(ones_w, 2, 1, 2, 1)

    outs = []
    for bi in range(_B):
        m1 = mask1x[bi]
        h = _fconv(_padzx(dense8f[bi]), w_in_f, sa_in, 1, 1, 25, 64,
                   maskx=m1)
        h = _fblock(h, wr['r1a1'][0], wr['r1a2'][0], wr['r1a1'][1], m1,
                    25, 64)
        h = _fblock(h, wr['r1b1'][0], wr['r1b2'][0], wr['r1b1'][1], m1,
                    25, 64)

        h = _fconv(h, wd2, sa_d2, 2, 2, 13, 32)
        m2 = _fconv(_padzx(mask1f[bi]), dil2_w, sa_dil2, 2, 2, 13, 32,
                    dilate=True)
        m2x = jnp.repeat(m2[1:14, 1:9, 1:33], 32, axis=-1)
        h = _fblock(h, wr['r2a1'][0], wr['r2a2'][0], wr['r2a1'][1], m2x,
                    13, 32)
        h = _fblock(h, wr['r2b1'][0], wr['r2b2'][0], wr['r2b1'][1], m2x,
                    13, 32)

        h = _fconv(h, wd3, sa_d3, 2, 2, 7, 16)
        m3 = _fconv(m2, dil3_w, sa_dil3, 2, 2, 7, 16, dilate=True)
        m3x = jnp.repeat(m3[1:8, 1:9, 1:17], 64, axis=-1)
        h = _fblock(h, wr['r3a1'][0], wr['r3a2'][0], wr['r3a1'][1], m3x,
                    7, 16)
        h = _fblock(h, wr['r3b1'][0], wr['r3b2'][0], wr['r3b1'][1], m3x,
                    7, 16)

        h = _fconv(h, wd4, sa_d4, 2, 2, 3, 8, bz=1)
        m4 = _fconv(m3, dil4_w, sa_dil4, 2, 2, 3, 8, bz=1, dilate=True)
        m4x = jnp.repeat(m4[1:4, 1:9, 1:9], 128, axis=-1)
        h = _fblock(h, wr['r4a1'][0], wr['r4a2'][0], wr['r4a1'][1], m4x,
                    3, 8)
        h = _fblock(h, wr['r4b1'][0], wr['r4b2'][0], wr['r4b1'][1], m4x,
                    3, 8)

        out = _fconv(h, wout, sa_out, 2, 1, 1, 8, bz=1, bx=1,
                     pad_out=False)
        outs.append(out)

    return jnp.stack(outs)


# speedup vs baseline: 1.4726x; 1.0867x over previous
import functools
import math

import numpy as np

import jax
import jax.numpy as jnp
from jax import lax
from jax.experimental import pallas as pl
from jax.experimental.pallas import tpu as pltpu
from jax.experimental.pallas import tpu_sc as plsc

_BNS = 1.0 / math.sqrt(1.0 + 1e-3)
_SPATIAL = (25, 64, 64)
_B = 2
_YB = 8


def _pcall(body, out_shape, scratch_shapes=(), interpret=False):
    return pl.pallas_call(body, out_shape=out_shape,
                          scratch_shapes=list(scratch_shapes),
                          interpret=interpret)


_NSITE = _B * 25 * 64 * 64
_TRASH = 128
_DR = 16
_NSH = _NSITE // 2
_HALF = _NSH + _TRASH


def _sc_scatter(idx0, idx1, feats16):
    NP = idx0.shape[0]
    ZB = 128
    CH = 512
    GR = 128
    n_chunks = NP // CH
    n_groups = CH // GR
    rows_t = _NSH // 16
    nz_full, nz_rem = divmod(rows_t, ZB)

    mesh = plsc.VectorSubcoreMesh(core_axis_name="c", subcore_axis_name="s")
    scratch = ([pltpu.VMEM_SHARED((_HALF, _DR), jnp.float32),
                pltpu.VMEM((ZB, _DR), jnp.float32)]
               + [pltpu.VMEM((GR,), jnp.int32) for _ in range(n_groups)]
               + [pltpu.VMEM((CH, _DR), jnp.float32),
                  pltpu.SemaphoreType.DMA])

    @functools.partial(pl.kernel, mesh=mesh,
                       out_type=jax.ShapeDtypeStruct((2 * _NSH, _DR),
                                                     jnp.float32),
                       scratch_types=scratch,
                       compiler_params=pltpu.CompilerParams(
                           use_tc_tiling_on_sc=False))
    def run(idx0_hbm, idx1_hbm, feats_hbm, out_hbm, shared, zbuf, *rest):
        idx_bufs = rest[:n_groups]
        rows_v = rest[n_groups]
        sem = rest[n_groups + 1]
        cid = lax.axis_index("c")
        sid = lax.axis_index("s")

        def zrow(r, carry):
            zbuf[r] = jnp.zeros((_DR,), jnp.float32)
            return carry
        lax.fori_loop(0, ZB, zrow, 0)
        base = sid * rows_t
        for k in range(nz_full):
            pltpu.sync_copy(zbuf, shared.at[pl.ds(base + k * ZB, ZB)])
        if nz_rem:
            pltpu.sync_copy(zbuf.at[pl.ds(0, nz_rem)],
                            shared.at[pl.ds(base + nz_full * ZB, nz_rem)])
        plsc.subcore_barrier()

        for c in range(2):
            @pl.when(jnp.logical_and(cid == c, sid == 0))
            def _scatter_phase(c=c):
                ih = idx0_hbm if c == 0 else idx1_hbm
                for ch in range(n_chunks):
                    pltpu.sync_copy(feats_hbm.at[pl.ds(ch * CH, CH)], rows_v)
                    for g in range(n_groups):
                        pltpu.sync_copy(ih.at[pl.ds(ch * CH + g * GR, GR)],
                                        idx_bufs[g])
                    for g in range(n_groups):
                        pltpu.async_copy(rows_v.at[pl.ds(g * GR, GR)],
                                         shared.at[idx_bufs[g]], sem).wait()
        plsc.subcore_barrier()
        pltpu.sync_copy(shared.at[pl.ds(base, rows_t)],
                        out_hbm.at[pl.ds(cid * _NSH + base, rows_t)])

    return run(idx0, idx1, feats16)


def _fold_w(w, gi, go, sy, py, scale=1.0):
    tz, ty, tx, ci, co = w.shape
    P = np.zeros((3, gi, go, ty), np.float32)
    for si, s in enumerate((-1, 0, 1)):
        for g_in in range(gi):
            for g_out in range(go):
                dy = g_in + s * gi - sy * g_out + py
                if 0 <= dy < ty:
                    P[si, g_in, g_out, dy] = scale
    wf = jnp.einsum('sghy,zyxio->szxgiho', P, w)
    wf = wf.reshape(3, tz, tx, gi * ci, go * co).astype(jnp.bfloat16)
    s_active = [si for si in range(3) if P[si].any()]
    return wf, s_active


def _taps(xp_ref, wf_ref, s_active, sz, sx, zo, xo, bz, bx, m, k, n, tz, tx):
    n_taps = len(s_active) * tz * tx
    s0 = s_active[0]

    def tap_body(t, acc):
        j = t // (tz * tx)
        dz = (t // tx) % tz
        dx = t % tx
        si = j + s0
        xs = xp_ref[pl.ds(dz + bz, sz * zo), pl.ds(si, _YB),
                    pl.ds(dx + bx, sx * xo), :]
        if sz > 1 or sx > 1:
            xs = xs.reshape(zo, sz, _YB, xo, sx, k)[:, 0, :, :, 0, :]
        return acc + jnp.dot(xs.reshape(m, k).astype(jnp.bfloat16),
                             wf_ref[si, dz, dx],
                             preferred_element_type=jnp.float32)

    return lax.fori_loop(0, n_taps, tap_body, jnp.zeros((m, n), jnp.float32))


def _store_padded(out_ref, val5, zo, xo, zr=2):
    z = jnp.float32(0.0)
    out_ref[pl.ds(0, 1)] = jnp.broadcast_to(z, out_ref.shape)[0:1]
    out_ref[pl.ds(zo + 1, zr)] = jnp.broadcast_to(z, out_ref.shape)[:zr]
    out_ref[:, pl.ds(0, 1)] = jnp.broadcast_to(z, out_ref.shape)[:, 0:1]
    out_ref[:, pl.ds(_YB + 1, 1)] = jnp.broadcast_to(z, out_ref.shape)[:, 0:1]
    out_ref[:, :, pl.ds(0, 1)] = jnp.broadcast_to(z, out_ref.shape)[:, :, 0:1]
    out_ref[:, :, pl.ds(xo + 1, 1)] = jnp.broadcast_to(
        z, out_ref.shape)[:, :, 0:1]
    out_ref[pl.ds(1, zo), pl.ds(1, _YB), pl.ds(1, xo), :] = val5


def _conv_core(src_ref, wf_ref, sa, sz, sx, zo, xo, bz=0, bx=0):
    _, tz, tx, k, n = wf_ref.shape
    m = zo * _YB * xo
    return _taps(src_ref, wf_ref, sa, sz, sx, zo, xo, bz, bx,
                 m, k, n, tz, tx)


def _interior(ref, zo, xo):
    return ref[pl.ds(1, zo), pl.ds(1, _YB), pl.ds(1, xo), :]


def _level1(xp, maskp, w_in_f, sa_in, wblk, sa1):
    zo, xo = 25, 64
    m = zo * _YB * xo

    def body(xp_ref, mask_ref, win_ref, wa1_ref, wa2_ref, wb1_ref, wb2_ref,
             out_ref, scr_ref):
        mi = mask_ref[...].reshape(m, 128)

        acc = _conv_core(xp_ref, win_ref, sa_in, 1, 1, zo, xo)
        _store_padded(out_ref, jnp.maximum(acc * mi * _BNS, 0.0)
                      .reshape(zo, _YB, xo, 128), zo, xo)
        for wc1, wc2 in ((wa1_ref, wa2_ref), (wb1_ref, wb2_ref)):
            acc = _conv_core(out_ref, wc1, sa1, 1, 1, zo, xo)
            _store_padded(scr_ref, jnp.maximum(acc * mi * _BNS, 0.0)
                          .reshape(zo, _YB, xo, 128), zo, xo, zr=1)
            acc = _conv_core(scr_ref, wc2, sa1, 1, 1, zo, xo)
            ident = _interior(out_ref, zo, xo).reshape(m, 128)
            _store_padded(out_ref,
                          jnp.maximum(acc * mi * _BNS + ident, 0.0)
                          .reshape(zo, _YB, xo, 128), zo, xo)

    shape = (zo + 3, _YB + 2, xo + 2, 128)
    sshape = (zo + 2, _YB + 2, xo + 2, 128)
    return _pcall(body, jax.ShapeDtypeStruct(shape, jnp.float32),
                  scratch_shapes=[pltpu.VMEM(sshape, jnp.float32)])(
                      xp, maskp, w_in_f, *wblk)


def _levels234(h1, maskp, wd, dil, wblk, wout, sa_out):
    dims = [(13, 32), (7, 16), (3, 8)]

    def body(*refs):
        (h1_ref, m1_ref, wd2_ref, wd3_ref, wd4_ref, dl2_ref, dl3_ref,
         dl4_ref, wa21, wa22, wb21, wb22, wa31, wa32, wb31, wb32,
         wa41, wa42, wb41, wb42, wout_ref, out_ref,
         h2_ref, s2_ref, m2_ref, h3_ref, s3_ref, m3_ref,
         h4_ref, s4_ref, m4_ref) = refs

        lvl = [
            (h1_ref, m1_ref, wd2_ref, dl2_ref, (wa21, wa22, wb21, wb22),
             h2_ref, s2_ref, m2_ref, dims[0], 0),
            (h2_ref, m2_ref, wd3_ref, dl3_ref, (wa31, wa32, wb31, wb32),
             h3_ref, s3_ref, m3_ref, dims[1], 0),
            (h3_ref, m3_ref, wd4_ref, dl4_ref, (wa41, wa42, wb41, wb42),
             h4_ref, s4_ref, m4_ref, dims[2], 1),
        ]
        for (hin, min_, wd_ref, dl_ref, (w1, w2, w3, w4), hout, scr, mout,
             (zo, xo), bz) in lvl:
            m = zo * _YB * xo
            acc = _conv_core(hin, wd_ref, [0, 1], 2, 2, zo, xo, bz=bz)
            _store_padded(hout, jnp.maximum(acc * _BNS, 0.0)
                          .reshape(zo, _YB, xo, 128), zo, xo)
            macc = _conv_core(min_, dl_ref, [0, 1], 2, 2, zo, xo, bz=bz)
            _store_padded(mout, (macc > 0.0).astype(jnp.float32)
                          .reshape(zo, _YB, xo, 128), zo, xo)
            mi = _interior(mout, zo, xo).reshape(m, 128)
            for wc1, wc2 in ((w1, w2), (w3, w4)):
                acc = _conv_core(hout, wc1, [0, 1, 2], 1, 1, zo, xo)
                _store_padded(scr, jnp.maximum(acc * mi * _BNS, 0.0)
                              .reshape(zo, _YB, xo, 128), zo, xo)
                acc = _conv_core(scr, wc2, [0, 1, 2], 1, 1, zo, xo)
                ident = _interior(hout, zo, xo).reshape(m, 128)
                _store_padded(hout,
                              jnp.maximum(acc * mi * _BNS + ident, 0.0)
                              .reshape(zo, _YB, xo, 128), zo, xo)

        acc = _conv_core(h4_ref, wout_ref, sa_out, 2, 1, 1, 8, bz=1, bx=1)
        out_ref[...] = jnp.maximum(acc * _BNS, 0.0).reshape(1, _YB, 8, 128)

    def pbuf(zo, xo):
        return (zo + 3, _YB + 2, xo + 2, 128)

    scratch = []
    for zo, xo in dims:
        scratch += [pltpu.VMEM(pbuf(zo, xo), jnp.float32)] * 3
    return _pcall(body, jax.ShapeDtypeStruct((1, _YB, 8, 128), jnp.float32),
                  scratch_shapes=scratch)(
                      h1, maskp, wd[0], wd[1], wd[2], dil[0], dil[1], dil[2],
                      *wblk, wout)


def kernel(voxel_features, voxel_coords, batch_size, params):
    Z, Y, X = _SPATIAL
    N = voxel_features.shape[0]
    p = params

    b = voxel_coords[:, 0] % batch_size
    z = voxel_coords[:, 1] % Z
    y = voxel_coords[:, 2] % Y
    x = voxel_coords[:, 3] % X

    feats16 = jnp.concatenate(
        [voxel_features,
         jnp.ones((N, 1), jnp.float32),
         jnp.zeros((N, _DR - 6), jnp.float32)], axis=1)
    idx = ((((b * Z + z) * _YB + y // 8) * X + x) * 8 + y % 8).astype(jnp.int32)
    npad = (-N) % 512
    pad_i = jnp.full((npad,), _NSH, jnp.int32)
    idx0 = jnp.concatenate(
        [jnp.where(idx < _NSH, idx, _NSH).astype(jnp.int32), pad_i])
    idx1 = jnp.concatenate(
        [jnp.where(idx >= _NSH, idx - _NSH, _NSH).astype(jnp.int32), pad_i])
    feats_p = jnp.concatenate([feats16, jnp.zeros((npad, _DR), jnp.float32)])
    table = _sc_scatter(idx0, idx1, feats_p)
    dense8f = table.reshape(_B, Z, _YB, X, 128)
    mask1x = jnp.repeat(dense8f[..., 5::16], 16, axis=-1)
    mask1xp = jnp.pad(mask1x, ((0, 0), (1, 2), (1, 1), (1, 1), (0, 0)))

    w_in16 = jnp.pad(p['w_in'], ((0, 0), (0, 0), (0, 0), (0, 11), (0, 0)))
    w_in_f, sa_in = _fold_w(w_in16, 8, 8, 1, 1)

    wblk1 = []
    sa1 = None
    for k_, g, ci in [('r1a1', 8, 16), ('r1a2', 8, 16), ('r1b1', 8, 16),
                      ('r1b2', 8, 16)]:
        wf, sa1 = _fold_w(p[k_], g, g, 1, 1)
        wblk1.append(wf)
    wblk234 = []
    for k_, g, ci in [('r2a1', 4, 32), ('r2a2', 4, 32), ('r2b1', 4, 32),
                      ('r2b2', 4, 32),
                      ('r3a1', 2, 64), ('r3a2', 2, 64), ('r3b1', 2, 64),
                      ('r3b2', 2, 64),
                      ('r4a1', 1, 128), ('r4a2', 1, 128), ('r4b1', 1, 128),
                      ('r4b2', 1, 128)]:
        wf, _ = _fold_w(p[k_], g, g, 1, 1)
        wblk234.append(wf)
    wd2, _ = _fold_w(p['w_d2'], 8, 4, 2, 1)
    wd3, _ = _fold_w(p['w_d3'], 4, 2, 2, 1)
    wd4, _ = _fold_w(p['w_d4'], 2, 1, 2, 1)
    wout, sa_out = _fold_w(p['w_out'], 1, 1, 1, 0)
    dil2, _ = _fold_w(jnp.ones((3, 3, 3, 16, 32), jnp.float32), 8, 4, 2, 1)
    dil3, _ = _fold_w(jnp.ones((3, 3, 3, 32, 64), jnp.float32), 4, 2, 2, 1)
    dil4, _ = _fold_w(jnp.ones((3, 3, 3, 64, 128), jnp.float32), 2, 1, 2, 1)

    outs = []
    for bi in range(_B):
        xp0 = jnp.pad(dense8f[bi], ((1, 2), (1, 1), (1, 1), (0, 0)))
        h1 = _level1(xp0, mask1x[bi], w_in_f, sa_in, wblk1, sa1)
        out = _levels234(h1, mask1xp[bi], (wd2, wd3, wd4),
                         (dil2, dil3, dil4), wblk234, wout, sa_out)
        outs.append(out)

    return jnp.stack(outs)
